# Initial kernel scaffold; baseline (speedup 1.0000x reference)
import functools

import jax
import jax.numpy as jnp
from jax import lax
from jax.experimental import pallas as pl
from jax.experimental.pallas import tpu as pltpu

BBLK = 256


def _cleanup_body(z_ref, cb_ref, out_ref):
    bblk = z_ref.shape[0]
    f_total, k_total, _ = cb_ref.shape
    acc = None
    for f in range(f_total):
        zf = z_ref[:, f, :]
        cbf = cb_ref[f]
        sims = lax.dot_general(
            zf, cbf, (((1,), (1,)), ((), ())),
            preferred_element_type=jnp.float32,
            precision=lax.Precision.HIGHEST,
        )
        idx = jnp.argmax(jnp.abs(sims), axis=1)
        onehot = (
            idx[:, None]
            == lax.broadcasted_iota(jnp.int32, (bblk, k_total), 1)
        ).astype(jnp.float32)
        wf = lax.dot_general(
            onehot, cbf, (((1,), (0,)), ((), ())),
            preferred_element_type=jnp.float32,
        )
        acc = wf if acc is None else acc * wf
    out_ref[...] = acc


@jax.jit
def kernel(z, codebooks):
    b, f, d = z.shape
    return pl.pallas_call(
        _cleanup_body,
        grid=(b // BBLK,),
        in_specs=[
            pl.BlockSpec((BBLK, f, d), lambda i: (i, 0, 0)),
            pl.BlockSpec(codebooks.shape, lambda i: (0, 0, 0)),
        ],
        out_specs=pl.BlockSpec((BBLK, d), lambda i: (i, 0)),
        out_shape=jax.ShapeDtypeStruct((b, d), jnp.float32),
        compiler_params=pltpu.CompilerParams(
            dimension_semantics=("arbitrary",),
        ),
    )(z, codebooks)


# speedup vs baseline: 7.8974x; 7.8974x over previous
import functools

import jax
import jax.numpy as jnp
from jax import lax
from jax.experimental import pallas as pl
from jax.experimental.pallas import tpu as pltpu

BBLK = 256


def _cleanup_body(z_ref, cb_ref, out_ref):
    bblk = z_ref.shape[0]
    f_total, k_total, _ = cb_ref.shape
    acc = None
    for f in range(f_total):
        zf = z_ref[:, f, :]
        cbf = cb_ref[f]
        sims = lax.dot_general(
            zf, cbf, (((1,), (1,)), ((), ())),
            preferred_element_type=jnp.float32,
        )
        idx = jnp.argmax(jnp.abs(sims), axis=1)
        onehot = (
            idx[:, None]
            == lax.broadcasted_iota(jnp.int32, (bblk, k_total), 1)
        ).astype(jnp.float32)
        wf = lax.dot_general(
            onehot, cbf, (((1,), (0,)), ((), ())),
            preferred_element_type=jnp.float32,
        )
        acc = wf if acc is None else acc * wf
    out_ref[...] = acc


@jax.jit
def kernel(z, codebooks):
    b, f, d = z.shape
    return pl.pallas_call(
        _cleanup_body,
        grid=(b // BBLK,),
        in_specs=[
            pl.BlockSpec((BBLK, f, d), lambda i: (i, 0, 0)),
            pl.BlockSpec(codebooks.shape, lambda i: (0, 0, 0)),
        ],
        out_specs=pl.BlockSpec((BBLK, d), lambda i: (i, 0)),
        out_shape=jax.ShapeDtypeStruct((b, d), jnp.float32),
        compiler_params=pltpu.CompilerParams(
            dimension_semantics=("arbitrary",),
        ),
    )(z, codebooks)


# speedup vs baseline: 7.9008x; 1.0004x over previous
import functools

import jax
import jax.numpy as jnp
from jax import lax
from jax.experimental import pallas as pl
from jax.experimental.pallas import tpu as pltpu

BBLK = 256


def _cleanup_body(z_ref, cb_ref, out_ref):
    bblk = z_ref.shape[0]
    f_total, k_total, _ = cb_ref.shape
    acc = None
    for f in range(f_total):
        zf = z_ref[:, f, :]
        cbf = cb_ref[f]
        sims = lax.dot_general(
            zf, cbf, (((1,), (1,)), ((), ())),
            preferred_element_type=jnp.float32,
        )
        idx = jnp.argmax(jnp.abs(sims), axis=1)
        onehot = (
            idx[:, None]
            == lax.broadcasted_iota(jnp.int32, (bblk, k_total), 1)
        ).astype(jnp.float32)
        wf = lax.dot_general(
            onehot.astype(jnp.bfloat16), cbf.astype(jnp.bfloat16),
            (((1,), (0,)), ((), ())),
            preferred_element_type=jnp.float32,
        )
        acc = wf if acc is None else acc * wf
    out_ref[...] = acc


@jax.jit
def kernel(z, codebooks):
    b, f, d = z.shape
    return pl.pallas_call(
        _cleanup_body,
        grid=(b // BBLK,),
        in_specs=[
            pl.BlockSpec((BBLK, f, d), lambda i: (i, 0, 0)),
            pl.BlockSpec(codebooks.shape, lambda i: (0, 0, 0)),
        ],
        out_specs=pl.BlockSpec((BBLK, d), lambda i: (i, 0)),
        out_shape=jax.ShapeDtypeStruct((b, d), jnp.float32),
        compiler_params=pltpu.CompilerParams(
            dimension_semantics=("arbitrary",),
        ),
    )(z, codebooks)
